# baseline (speedup 1.0000x reference)
import jax
import jax.numpy as jnp
from jax.experimental import pallas as pl
from jax.experimental.pallas import tpu as pltpu
from jax.experimental.pallas import tpu_sc as plsc

_GATHER_WINDOW = 128
_BV = 2048
_NO = 4
_NS = 2
_NCORES = 2


def _sc_gather_packed(packed_table, packed_idx):
    n = packed_idx.shape[0]
    idx2d = packed_idx.reshape(1, n)
    mesh = plsc.VectorSubcoreMesh(core_axis_name="core", subcore_axis_name="subcore")

    @pl.kernel(
        out_type=jax.ShapeDtypeStruct((n, packed_table.shape[1]), packed_table.dtype),
        mesh=mesh,
    )
    def gather_kernel(table_hbm, idx_hbm, out_hbm):
        def body(idx_vmem, out_vmem):
            pltpu.sync_copy(table_hbm.at[idx_vmem.at[0]], out_vmem)

        pltpu.emit_pipeline(
            body,
            grid=(n // _GATHER_WINDOW,),
            in_specs=[pl.BlockSpec((1, _GATHER_WINDOW), index_map=lambda i: (0, i))],
            out_specs=[
                pl.BlockSpec(
                    (_GATHER_WINDOW, packed_table.shape[1]),
                    index_map=lambda i: (i, 0),
                )
            ],
            core_axis_name="subcore",
            dimension_semantics=(pltpu.PARALLEL,),
        )(idx_hbm, out_hbm)

    return gather_kernel(packed_table, idx2d)


def _tc_project(h_packed, parity, W, b2d):
    batch = h_packed.shape[0]
    dim, vocab = W.shape
    nb_full, rem = divmod(vocab, _BV)
    per_core = nb_full // _NCORES
    rows = batch // _NS
    mesh = pltpu.create_tensorcore_mesh("core", num_cores=_NCORES)

    @pl.kernel(
        out_type=jax.ShapeDtypeStruct((batch, vocab), jnp.float32),
        mesh=mesh,
        scratch_types=[
            pltpu.VMEM((batch, 2 * dim), jnp.float32),
            pltpu.VMEM((batch, 1), jnp.int32),
            pltpu.VMEM((batch, dim), jnp.float32),
            pltpu.VMEM((2, dim, _BV), jnp.float32),
            pltpu.VMEM((2, 1, _BV), jnp.float32),
            pltpu.VMEM((_NO, batch, _BV), jnp.float32),
            pltpu.VMEM((dim, max(rem, 1)), jnp.float32),
            pltpu.VMEM((1, max(rem, 1)), jnp.float32),
            pltpu.VMEM((batch, max(rem, 1)), jnp.float32),
            pltpu.SemaphoreType.DMA,
            pltpu.SemaphoreType.DMA,
            pltpu.SemaphoreType.DMA((2,)),
            pltpu.SemaphoreType.DMA((2,)),
            pltpu.SemaphoreType.DMA((_NO, _NS)),
            pltpu.SemaphoreType.DMA((3,)),
        ],
    )
    def project(hp_hbm, par_hbm, w_hbm, b_hbm, o_hbm,
                hp_v, par_v, h_v, w_bufs, b_bufs, o_bufs,
                w_tail, b_tail, o_tail,
                hp_sem, par_sem, w_sems, b_sems, o_sems, tail_sems):
        core = jax.lax.axis_index("core")
        base = core * (per_core * _BV)

        def start_wb(k):
            sw = k % 2
            pltpu.make_async_copy(
                w_hbm.at[:, pl.ds(base + k * _BV, _BV)], w_bufs.at[sw],
                w_sems.at[sw],
            ).start()
            pltpu.make_async_copy(
                b_hbm.at[:, pl.ds(base + k * _BV, _BV)], b_bufs.at[sw],
                b_sems.at[sw],
            ).start()

        pltpu.make_async_copy(hp_hbm, hp_v, hp_sem).start()
        pltpu.make_async_copy(par_hbm, par_v, par_sem).start()
        start_wb(0)
        start_wb(1)

        is_tail_core = core == _NCORES - 1
        if rem:
            @pl.when(is_tail_core)
            def _():
                pltpu.make_async_copy(
                    w_hbm.at[:, pl.ds(nb_full * _BV, rem)], w_tail,
                    tail_sems.at[0],
                ).start()
                pltpu.make_async_copy(
                    b_hbm.at[:, pl.ds(nb_full * _BV, rem)], b_tail,
                    tail_sems.at[1],
                ).start()

        pltpu.make_async_copy(hp_hbm, hp_v, hp_sem).wait()
        pltpu.make_async_copy(par_hbm, par_v, par_sem).wait()
        h_v[...] = jnp.where(par_v[...] != 0, hp_v[:, dim:], hp_v[:, :dim])

        outstanding = {}
        for k in range(per_core):
            sw, so = k % 2, k % _NO
            pltpu.make_async_copy(
                w_hbm.at[:, pl.ds(base + k * _BV, _BV)], w_bufs.at[sw],
                w_sems.at[sw],
            ).wait()
            pltpu.make_async_copy(
                b_hbm.at[:, pl.ds(base + k * _BV, _BV)], b_bufs.at[sw],
                b_sems.at[sw],
            ).wait()
            if so in outstanding:
                for cp in outstanding.pop(so):
                    cp.wait()
            o_bufs[so] = (
                jnp.dot(h_v[...], w_bufs[sw], preferred_element_type=jnp.float32)
                + b_bufs[sw]
            )
            copies = []
            for s in range(_NS):
                cp = pltpu.make_async_copy(
                    o_bufs.at[so, pl.ds(s * rows, rows), :],
                    o_hbm.at[pl.ds(s * rows, rows), pl.ds(base + k * _BV, _BV)],
                    o_sems.at[so, s],
                )
                cp.start()
                copies.append(cp)
            outstanding[so] = copies
            if k + 2 < per_core:
                start_wb(k + 2)
        if rem:
            @pl.when(is_tail_core)
            def _():
                pltpu.make_async_copy(
                    w_hbm.at[:, pl.ds(nb_full * _BV, rem)], w_tail,
                    tail_sems.at[0],
                ).wait()
                pltpu.make_async_copy(
                    b_hbm.at[:, pl.ds(nb_full * _BV, rem)], b_tail,
                    tail_sems.at[1],
                ).wait()
                o_tail[...] = (
                    jnp.dot(h_v[...], w_tail[...],
                            preferred_element_type=jnp.float32)
                    + b_tail[...]
                )
                pltpu.make_async_copy(
                    o_tail, o_hbm.at[:, pl.ds(nb_full * _BV, rem)],
                    tail_sems.at[2],
                ).start()
                pltpu.make_async_copy(
                    o_tail, o_hbm.at[:, pl.ds(nb_full * _BV, rem)],
                    tail_sems.at[2],
                ).wait()
        for copies in outstanding.values():
            for cp in copies:
                cp.wait()

    return project(h_packed, parity, W, b2d)


def kernel(input_ids, embed_table, W, b):
    vocab_rows, dim = embed_table.shape
    packed_table = embed_table.reshape(vocab_rows // 2, 2 * dim)
    h_packed = _sc_gather_packed(packed_table, input_ids // 2)
    parity = (input_ids % 2).astype(jnp.int32).reshape(-1, 1)
    return _tc_project(h_packed, parity, W, b.reshape(1, -1))


# speedup vs baseline: 2.1067x; 2.1067x over previous
import jax
import jax.numpy as jnp
from jax.experimental import pallas as pl
from jax.experimental.pallas import tpu as pltpu
from jax.experimental.pallas import tpu_sc as plsc

_GATHER_WINDOW = 128
_BV = 2048


def _sc_gather_packed(packed_table, packed_idx):
    n = packed_idx.shape[0]
    idx2d = packed_idx.reshape(1, n)
    mesh = plsc.VectorSubcoreMesh(core_axis_name="core", subcore_axis_name="subcore")

    @pl.kernel(
        out_type=jax.ShapeDtypeStruct((n, packed_table.shape[1]), packed_table.dtype),
        mesh=mesh,
    )
    def gather_kernel(table_hbm, idx_hbm, out_hbm):
        def body(idx_vmem, out_vmem):
            pltpu.sync_copy(table_hbm.at[idx_vmem.at[0]], out_vmem)

        pltpu.emit_pipeline(
            body,
            grid=(n // _GATHER_WINDOW,),
            in_specs=[pl.BlockSpec((1, _GATHER_WINDOW), index_map=lambda i: (0, i))],
            out_specs=[
                pl.BlockSpec(
                    (_GATHER_WINDOW, packed_table.shape[1]),
                    index_map=lambda i: (i, 0),
                )
            ],
            core_axis_name="subcore",
            dimension_semantics=(pltpu.PARALLEL,),
        )(idx_hbm, out_hbm)

    return gather_kernel(packed_table, idx2d)


def _tc_project_t(h_packed, parity, W, bcol):
    batch = h_packed.shape[0]
    dim, vocab = W.shape
    grid = pl.cdiv(vocab, _BV)

    def body(hp_ref, par_ref, w_ref, b_ref, o_ref):
        h = jnp.where(par_ref[...] != 0, hp_ref[:, dim:], hp_ref[:, :dim])
        o_ref[...] = (
            jax.lax.dot_general(
                w_ref[...], h,
                (((0,), (1,)), ((), ())),
                preferred_element_type=jnp.float32,
            )
            + b_ref[...]
        )

    return pl.pallas_call(
        body,
        grid=(grid,),
        in_specs=[
            pl.BlockSpec((batch, 2 * dim), lambda k: (0, 0)),
            pl.BlockSpec((batch, 1), lambda k: (0, 0)),
            pl.BlockSpec((dim, _BV), lambda k: (0, k)),
            pl.BlockSpec((_BV, 1), lambda k: (k, 0)),
        ],
        out_specs=pl.BlockSpec((_BV, batch), lambda k: (k, 0)),
        out_shape=jax.ShapeDtypeStruct((vocab, batch), jnp.float32),
        compiler_params=pltpu.CompilerParams(
            dimension_semantics=("arbitrary",),
        ),
    )(h_packed, parity, W, bcol)


def kernel(input_ids, embed_table, W, b):
    vocab_rows, dim = embed_table.shape
    packed_table = embed_table.reshape(vocab_rows // 2, 2 * dim)
    h_packed = _sc_gather_packed(packed_table, input_ids // 2)
    parity = (input_ids % 2).astype(jnp.int32).reshape(-1, 1)
    logits_t = _tc_project_t(h_packed, parity, W, b.reshape(-1, 1))
    return jnp.transpose(logits_t)


# speedup vs baseline: 2.6490x; 1.2574x over previous
import jax
import jax.numpy as jnp
from jax.experimental import pallas as pl
from jax.experimental.pallas import tpu as pltpu
from jax.experimental.pallas import tpu_sc as plsc

_GATHER_WINDOW = 128
_BV = 2048


def _sc_gather_packed(packed_table, packed_idx):
    n = packed_idx.shape[0]
    idx2d = packed_idx.reshape(1, n)
    mesh = plsc.VectorSubcoreMesh(core_axis_name="core", subcore_axis_name="subcore")

    @pl.kernel(
        out_type=jax.ShapeDtypeStruct((n, packed_table.shape[1]), packed_table.dtype),
        mesh=mesh,
    )
    def gather_kernel(table_hbm, idx_hbm, out_hbm):
        def body(idx_vmem, out_vmem):
            pltpu.sync_copy(table_hbm.at[idx_vmem.at[0]], out_vmem)

        pltpu.emit_pipeline(
            body,
            grid=(n // _GATHER_WINDOW,),
            in_specs=[pl.BlockSpec((1, _GATHER_WINDOW), index_map=lambda i: (0, i))],
            out_specs=[
                pl.BlockSpec(
                    (_GATHER_WINDOW, packed_table.shape[1]),
                    index_map=lambda i: (i, 0),
                )
            ],
            core_axis_name="subcore",
            dimension_semantics=(pltpu.PARALLEL,),
        )(idx_hbm, out_hbm)

    return gather_kernel(packed_table, idx2d)


def _tc_project_t(h_packed, parity, W, bcol):
    batch = h_packed.shape[0]
    dim, vocab = W.shape
    grid = pl.cdiv(vocab, _BV)

    def body(hp_ref, par_ref, w_ref, b_ref, o_ref):
        h = jnp.where(par_ref[...] != 0, hp_ref[:, dim:], hp_ref[:, :dim])
        w_aug = jnp.concatenate([w_ref[...], b_ref[...]], axis=0)
        h_aug = jnp.concatenate(
            [h, jnp.ones((batch, 1), jnp.float32)], axis=1
        )
        o_ref[...] = jax.lax.dot_general(
            w_aug, h_aug,
            (((0,), (1,)), ((), ())),
            preferred_element_type=jnp.float32,
        )

    return pl.pallas_call(
        body,
        grid=(grid,),
        in_specs=[
            pl.BlockSpec((batch, 2 * dim), lambda k: (0, 0)),
            pl.BlockSpec((batch, 1), lambda k: (0, 0)),
            pl.BlockSpec((dim, _BV), lambda k: (0, k)),
            pl.BlockSpec((1, _BV), lambda k: (0, k)),
        ],
        out_specs=pl.BlockSpec((_BV, batch), lambda k: (k, 0)),
        out_shape=jax.ShapeDtypeStruct((vocab, batch), jnp.float32),
        compiler_params=pltpu.CompilerParams(
            dimension_semantics=("arbitrary",),
        ),
    )(h_packed, parity, W, bcol)


def kernel(input_ids, embed_table, W, b):
    vocab_rows, dim = embed_table.shape
    packed_table = embed_table.reshape(vocab_rows // 2, 2 * dim)
    h_packed = _sc_gather_packed(packed_table, input_ids // 2)
    parity = (input_ids % 2).astype(jnp.int32).reshape(-1, 1)
    logits_t = _tc_project_t(h_packed, parity, W, b.reshape(1, -1))
    return jnp.transpose(logits_t)


# speedup vs baseline: 2.6893x; 1.0152x over previous
import jax
import jax.numpy as jnp
from jax.experimental import pallas as pl
from jax.experimental.pallas import tpu as pltpu
from jax.experimental.pallas import tpu_sc as plsc

_GATHER_WINDOW = 128
_BV = 4096


def _sc_gather_packed(packed_table, packed_idx):
    n = packed_idx.shape[0]
    idx2d = packed_idx.reshape(1, n)
    mesh = plsc.VectorSubcoreMesh(core_axis_name="core", subcore_axis_name="subcore")

    @pl.kernel(
        out_type=jax.ShapeDtypeStruct((n, packed_table.shape[1]), packed_table.dtype),
        mesh=mesh,
    )
    def gather_kernel(table_hbm, idx_hbm, out_hbm):
        def body(idx_vmem, out_vmem):
            pltpu.sync_copy(table_hbm.at[idx_vmem.at[0]], out_vmem)

        pltpu.emit_pipeline(
            body,
            grid=(n // _GATHER_WINDOW,),
            in_specs=[pl.BlockSpec((1, _GATHER_WINDOW), index_map=lambda i: (0, i))],
            out_specs=[
                pl.BlockSpec(
                    (_GATHER_WINDOW, packed_table.shape[1]),
                    index_map=lambda i: (i, 0),
                )
            ],
            core_axis_name="subcore",
            dimension_semantics=(pltpu.PARALLEL,),
        )(idx_hbm, out_hbm)

    return gather_kernel(packed_table, idx2d)


def _tc_project_t(h_packed, parity, W, bcol):
    batch = h_packed.shape[0]
    dim, vocab = W.shape
    grid = pl.cdiv(vocab, _BV)

    def body(hp_ref, par_ref, w_ref, b_ref, o_ref):
        h = jnp.where(par_ref[...] != 0, hp_ref[:, dim:], hp_ref[:, :dim])
        w_aug = jnp.concatenate([w_ref[...], b_ref[...]], axis=0)
        h_aug = jnp.concatenate(
            [h, jnp.ones((batch, 1), jnp.float32)], axis=1
        )
        o_ref[...] = jax.lax.dot_general(
            w_aug, h_aug,
            (((0,), (1,)), ((), ())),
            preferred_element_type=jnp.float32,
        )

    return pl.pallas_call(
        body,
        grid=(grid,),
        in_specs=[
            pl.BlockSpec((batch, 2 * dim), lambda k: (0, 0)),
            pl.BlockSpec((batch, 1), lambda k: (0, 0)),
            pl.BlockSpec((dim, _BV), lambda k: (0, k)),
            pl.BlockSpec((1, _BV), lambda k: (0, k)),
        ],
        out_specs=pl.BlockSpec((_BV, batch), lambda k: (k, 0)),
        out_shape=jax.ShapeDtypeStruct((vocab, batch), jnp.float32),
        compiler_params=pltpu.CompilerParams(
            dimension_semantics=("arbitrary",),
        ),
    )(h_packed, parity, W, bcol)


def kernel(input_ids, embed_table, W, b):
    vocab_rows, dim = embed_table.shape
    packed_table = embed_table.reshape(vocab_rows // 2, 2 * dim)
    h_packed = _sc_gather_packed(packed_table, input_ids // 2)
    parity = (input_ids % 2).astype(jnp.int32).reshape(-1, 1)
    logits_t = _tc_project_t(h_packed, parity, W, b.reshape(1, -1))
    return jnp.transpose(logits_t)
